# Initial kernel scaffold; baseline (speedup 1.0000x reference)
import jax
import jax.numpy as jnp
from jax.experimental import pallas as pl


def kernel(x, edge_attr, edge_index, W1_0, b1_0, g_0, be_0, W2_0, b2_0, W1_1, b1_1, g_1, be_1, W2_1, b2_1, W1_2, b1_2, g_2, be_2, W2_2, b2_2, Wc, bc):
    raise NotImplementedError("write your pallas kernel here")


# speedup vs baseline: 2.8222x; 2.8222x over previous
import functools

import jax
import jax.numpy as jnp
from jax import lax
from jax.experimental import pallas as pl
from jax.experimental.pallas import tpu as pltpu
from jax.experimental.pallas import tpu_sc as plsc

_N = 10000
_DH = 128
_E = 320000
_EPS = 1e-5

_NC = 2
_NS = 16
_NW = _NC * _NS
_CH = 128
_CPT = 80
_EPAD = _NW * _CPT * _CH
_NROW_PAD = 10240
_RPT = _NROW_PAD // _NS


def _build_sc_agg():
    mesh = plsc.VectorSubcoreMesh(core_axis_name="c", subcore_axis_name="s")

    @functools.partial(
        pl.kernel,
        out_type=jax.ShapeDtypeStruct((_NC, _NROW_PAD, _DH), jnp.float32),
        mesh=mesh,
        scratch_types=[
            pltpu.VMEM((_CPT, _CH), jnp.int32),
            pltpu.VMEM((_CPT, _CH), jnp.int32),
            pltpu.VMEM((_CH, _DH), jnp.float32),
            pltpu.VMEM_SHARED((_NROW_PAD, _DH), jnp.float32),
            pltpu.SemaphoreType.DMA,
        ],
    )
    def sc_agg(h_hbm, src_hbm, dst_hbm, zeros_hbm, out_hbm,
               srcv, dstv, rows, agg, sem):
        c = lax.axis_index("c")
        s = lax.axis_index("s")
        w = c * _NS + s
        pltpu.sync_copy(src_hbm.at[pl.ds(w * _CPT, _CPT)], srcv)
        pltpu.sync_copy(dst_hbm.at[pl.ds(w * _CPT, _CPT)], dstv)
        pltpu.sync_copy(zeros_hbm, rows)
        r0 = s * _RPT
        for zi in range(_RPT // _CH):
            pltpu.sync_copy(rows, agg.at[pl.ds(r0 + zi * _CH, _CH)])
        plsc.subcore_barrier()

        def chunk(j, carry):
            pltpu.async_copy(h_hbm.at[srcv.at[j]], rows, sem).wait()
            pltpu.sync_copy(rows, agg.at[dstv.at[j]], add=True)
            return carry

        lax.fori_loop(0, _CPT, chunk, 0)
        plsc.subcore_barrier()
        pltpu.sync_copy(agg.at[pl.ds(r0, _RPT)], out_hbm.at[c, pl.ds(r0, _RPT)])

    return sc_agg


def _tc_mid(h, agg2, W1, b1, g, be, W2, b2):
    def body(h_ref, a_ref, w1, b1r, gr, ber, w2, b2r, o_ref):
        z = h_ref[...] + a_ref[0, :_N, :] + a_ref[1, :_N, :]
        t = jnp.dot(z, w1[...], preferred_element_type=jnp.float32) + b1r[...]
        mu = jnp.mean(t, axis=0, keepdims=True)
        var = jnp.mean(jnp.square(t - mu), axis=0, keepdims=True)
        t = (t - mu) / jnp.sqrt(var + _EPS) * gr[...] + ber[...]
        t = jnp.maximum(t, 0.0)
        o = jnp.dot(t, w2[...], preferred_element_type=jnp.float32) + b2r[...]
        o_ref[...] = jnp.maximum(o, 0.0)

    return pl.pallas_call(
        body, out_shape=jax.ShapeDtypeStruct((_N, _DH), jnp.float32),
    )(h, agg2, W1, b1.reshape(1, -1), g.reshape(1, -1), be.reshape(1, -1),
      W2, b2.reshape(1, -1))


def _tc_last(h, agg2, W1, b1, g, be, W2, b2, Wc, bc):
    d_out = Wc.shape[1]

    def body(h_ref, a_ref, w1, b1r, gr, ber, w2, b2r, wc, bcr, o_ref):
        z = h_ref[...] + a_ref[0, :_N, :] + a_ref[1, :_N, :]
        t = jnp.dot(z, w1[...], preferred_element_type=jnp.float32) + b1r[...]
        mu = jnp.mean(t, axis=0, keepdims=True)
        var = jnp.mean(jnp.square(t - mu), axis=0, keepdims=True)
        t = (t - mu) / jnp.sqrt(var + _EPS) * gr[...] + ber[...]
        t = jnp.maximum(t, 0.0)
        o = jnp.dot(t, w2[...], preferred_element_type=jnp.float32) + b2r[...]
        hh = jnp.maximum(o, 0.0)
        o_ref[...] = jnp.dot(hh, wc[...], preferred_element_type=jnp.float32) + bcr[...]

    return pl.pallas_call(
        body, out_shape=jax.ShapeDtypeStruct((_N, d_out), jnp.float32),
    )(h, agg2, W1, b1.reshape(1, -1), g.reshape(1, -1), be.reshape(1, -1),
      W2, b2.reshape(1, -1), Wc, bc.reshape(1, -1))


def kernel(x, edge_attr, edge_index,
           W1_0, b1_0, g_0, be_0, W2_0, b2_0,
           W1_1, b1_1, g_1, be_1, W2_1, b2_1,
           W1_2, b1_2, g_2, be_2, W2_2, b2_2,
           Wc, bc):
    del edge_attr
    src = edge_index[0]
    dst = edge_index[1]
    pad = _EPAD - _E
    src2d = jnp.concatenate(
        [src, jnp.zeros((pad,), jnp.int32)]).reshape(_NW * _CPT, _CH)
    dst2d = jnp.concatenate(
        [dst, jnp.full((pad,), _N, jnp.int32)]).reshape(_NW * _CPT, _CH)
    zeros = jnp.zeros((_CH, _DH), jnp.float32)

    sc_agg = _build_sc_agg()
    params = [
        (W1_0, b1_0, g_0, be_0, W2_0, b2_0),
        (W1_1, b1_1, g_1, be_1, W2_1, b2_1),
    ]
    h = x
    for (W1, b1, g, be, W2, b2) in params:
        agg2 = sc_agg(h, src2d, dst2d, zeros)
        h = _tc_mid(h, agg2, W1, b1, g, be, W2, b2)
    agg2 = sc_agg(h, src2d, dst2d, zeros)
    return _tc_last(h, agg2, W1_2, b1_2, g_2, be_2, W2_2, b2_2, Wc, bc)


# speedup vs baseline: 2.8413x; 1.0068x over previous
import functools

import jax
import jax.numpy as jnp
from jax import lax
from jax.experimental import pallas as pl
from jax.experimental.pallas import tpu as pltpu
from jax.experimental.pallas import tpu_sc as plsc

_N = 10000
_DH = 128
_E = 320000
_EPS = 1e-5

_NC = 2
_NS = 16
_NW = _NC * _NS
_CH = 128
_CPT = 80
_EPAD = _NW * _CPT * _CH
_NROW_PAD = 10112
_RPT = _NROW_PAD // _NS
_PH = 2
_CPP = _CPT // _PH


def _build_sc_agg():
    mesh = plsc.VectorSubcoreMesh(core_axis_name="c", subcore_axis_name="s")

    @functools.partial(
        pl.kernel,
        out_type=jax.ShapeDtypeStruct((_NC, _NROW_PAD, _DH), jnp.float32),
        mesh=mesh,
        scratch_types=[
            pltpu.VMEM((_CPP, _CH), jnp.int32),
            pltpu.VMEM((_CPP, _CH), jnp.int32),
            pltpu.VMEM((_CH, _DH), jnp.float32),
            pltpu.VMEM((_CH, _DH), jnp.float32),
            pltpu.VMEM_SHARED((_NROW_PAD, _DH), jnp.float32),
            pltpu.SemaphoreType.DMA,
            pltpu.SemaphoreType.DMA,
            pltpu.SemaphoreType.DMA,
            pltpu.SemaphoreType.DMA,
        ],
    )
    def sc_agg(h_hbm, src_hbm, dst_hbm, zeros_hbm, out_hbm,
               srcv, dstv, rowsA, rowsB, agg, gsA, gsB, ssA, ssB):
        c = lax.axis_index("c")
        s = lax.axis_index("s")
        w = c * _NS + s
        pltpu.sync_copy(zeros_hbm, rowsA)
        r0 = s * _RPT
        nfull = _RPT // _CH
        for zi in range(nfull):
            pltpu.sync_copy(rowsA, agg.at[pl.ds(r0 + zi * _CH, _CH)])
        rem = _RPT - nfull * _CH
        if rem:
            pltpu.sync_copy(rowsA.at[pl.ds(0, rem)],
                            agg.at[pl.ds(r0 + nfull * _CH, rem)])
        plsc.subcore_barrier()

        def wait_g(buf, sem):
            pltpu.make_async_copy(h_hbm.at[srcv.at[0]], buf, sem).wait()

        def wait_s(buf, sem):
            pltpu.make_async_copy(buf, agg.at[dstv.at[0]], sem).wait()

        for p in range(_PH):
            base = w * _CPT + p * _CPP
            pltpu.sync_copy(src_hbm.at[pl.ds(base, _CPP)], srcv)
            pltpu.sync_copy(dst_hbm.at[pl.ds(base, _CPP)], dstv)
            pltpu.async_copy(h_hbm.at[srcv.at[0]], rowsA, gsA)

            def step(j2, carry):
                j = 2 * j2
                wait_g(rowsA, gsA)

                @pl.when(j2 > 0)
                def _():
                    wait_s(rowsB, ssB)

                pltpu.async_copy(h_hbm.at[srcv.at[j + 1]], rowsB, gsB)
                pltpu.async_copy(rowsA, agg.at[dstv.at[j]], ssA, add=True)
                wait_g(rowsB, gsB)
                wait_s(rowsA, ssA)

                @pl.when(j + 2 < _CPP)
                def _():
                    pltpu.async_copy(h_hbm.at[srcv.at[j + 2]], rowsA, gsA)

                pltpu.async_copy(rowsB, agg.at[dstv.at[j + 1]], ssB, add=True)
                return carry

            lax.fori_loop(0, _CPP // 2, step, 0)
            wait_s(rowsB, ssB)
        plsc.subcore_barrier()
        pltpu.sync_copy(agg.at[pl.ds(r0, _RPT)], out_hbm.at[c, pl.ds(r0, _RPT)])

    return sc_agg


def _tc_mid(h, agg2, W1, b1, g, be, W2, b2):
    def body(h_ref, a_ref, w1, b1r, gr, ber, w2, b2r, o_ref):
        z = h_ref[...] + a_ref[0, :_N, :] + a_ref[1, :_N, :]
        t = jnp.dot(z, w1[...], preferred_element_type=jnp.float32) + b1r[...]
        mu = jnp.mean(t, axis=0, keepdims=True)
        var = jnp.mean(jnp.square(t - mu), axis=0, keepdims=True)
        t = (t - mu) / jnp.sqrt(var + _EPS) * gr[...] + ber[...]
        t = jnp.maximum(t, 0.0)
        o = jnp.dot(t, w2[...], preferred_element_type=jnp.float32) + b2r[...]
        o_ref[...] = jnp.maximum(o, 0.0)

    return pl.pallas_call(
        body, out_shape=jax.ShapeDtypeStruct((_N, _DH), jnp.float32),
    )(h, agg2, W1, b1.reshape(1, -1), g.reshape(1, -1), be.reshape(1, -1),
      W2, b2.reshape(1, -1))


def _tc_last(h, agg2, W1, b1, g, be, W2, b2, Wc, bc):
    d_out = Wc.shape[1]

    def body(h_ref, a_ref, w1, b1r, gr, ber, w2, b2r, wc, bcr, o_ref):
        z = h_ref[...] + a_ref[0, :_N, :] + a_ref[1, :_N, :]
        t = jnp.dot(z, w1[...], preferred_element_type=jnp.float32) + b1r[...]
        mu = jnp.mean(t, axis=0, keepdims=True)
        var = jnp.mean(jnp.square(t - mu), axis=0, keepdims=True)
        t = (t - mu) / jnp.sqrt(var + _EPS) * gr[...] + ber[...]
        t = jnp.maximum(t, 0.0)
        o = jnp.dot(t, w2[...], preferred_element_type=jnp.float32) + b2r[...]
        hh = jnp.maximum(o, 0.0)
        o_ref[...] = jnp.dot(hh, wc[...], preferred_element_type=jnp.float32) + bcr[...]

    return pl.pallas_call(
        body, out_shape=jax.ShapeDtypeStruct((_N, d_out), jnp.float32),
    )(h, agg2, W1, b1.reshape(1, -1), g.reshape(1, -1), be.reshape(1, -1),
      W2, b2.reshape(1, -1), Wc, bc.reshape(1, -1))


def kernel(x, edge_attr, edge_index,
           W1_0, b1_0, g_0, be_0, W2_0, b2_0,
           W1_1, b1_1, g_1, be_1, W2_1, b2_1,
           W1_2, b1_2, g_2, be_2, W2_2, b2_2,
           Wc, bc):
    del edge_attr
    src = edge_index[0]
    dst = edge_index[1]
    pad = _EPAD - _E
    src2d = jnp.concatenate(
        [src, jnp.zeros((pad,), jnp.int32)]).reshape(_NW * _CPT, _CH)
    dst2d = jnp.concatenate(
        [dst, jnp.full((pad,), _N, jnp.int32)]).reshape(_NW * _CPT, _CH)
    zeros = jnp.zeros((_CH, _DH), jnp.float32)

    sc_agg = _build_sc_agg()
    params = [
        (W1_0, b1_0, g_0, be_0, W2_0, b2_0),
        (W1_1, b1_1, g_1, be_1, W2_1, b2_1),
    ]
    h = x
    for (W1, b1, g, be, W2, b2) in params:
        agg2 = sc_agg(h, src2d, dst2d, zeros)
        h = _tc_mid(h, agg2, W1, b1, g, be, W2, b2)
    agg2 = sc_agg(h, src2d, dst2d, zeros)
    return _tc_last(h, agg2, W1_2, b1_2, g_2, be_2, W2_2, b2_2, Wc, bc)
